# baseline (speedup 1.0000x reference)
import jax
import jax.numpy as jnp
from jax.experimental import pallas as pl
from jax.experimental.pallas import tpu as pltpu

_N = 32768
_D = 256
_K = 1024
_BN = 4096
_TR = 128
_C = 128


def _kmeans_block(x_ref, v_ref, o_ref, v2s_ref, v2n_ref, xv2_ref):
    @pl.when(pl.program_id(0) == 0)
    def _prep():
        v = v_ref[...]
        v2s_ref[...] = v * 2.0
        v2n_ref[...] = jnp.sum(v * v, axis=1)[None, :]

    v2 = v2n_ref[...]
    xv2_ref[...] = jax.lax.dot_general(
        x_ref[...], v2s_ref[...], (((1,), (1,)), ((), ())),
        preferred_element_type=jnp.float32)

    def _tile(ti, carry):
        rows = pl.ds(ti * _TR, _TR)
        x = x_ref[rows, :]
        x2 = jnp.sum(x * x, axis=1, keepdims=True)
        xv2 = xv2_ref[rows, :]
        m = (x2 + v2[:, :_C]) - xv2[:, :_C]
        ci = jnp.zeros(m.shape, jnp.int32)
        for i in range(1, _K // _C):
            t = (x2 + v2[:, i * _C:(i + 1) * _C]) - xv2[:, i * _C:(i + 1) * _C]
            lt = t < m
            ci = jnp.where(lt, i, ci)
            m = jnp.minimum(m, t)
        lanes = jax.lax.broadcasted_iota(jnp.int32, m.shape, 1)
        mv = jnp.min(m, axis=1, keepdims=True)
        col = ci * _C + lanes
        first = jnp.min(jnp.where(m == mv, col, _K), axis=1, keepdims=True)
        dmin = jnp.sqrt(jnp.maximum(mv, 1e-12))
        cols = jax.lax.broadcasted_iota(jnp.int32, (_TR, _K), 1)
        o_ref[rows, :] = jnp.where(cols == first, dmin, 0.0)
        return carry

    jax.lax.fori_loop(0, _BN // _TR, _tile, 0)


@jax.jit
def kernel(X, V):
    grid = (_N // _BN,)
    return pl.pallas_call(
        _kmeans_block,
        grid=grid,
        in_specs=[
            pl.BlockSpec((_BN, _D), lambda i: (i, 0)),
            pl.BlockSpec((_K, _D), lambda i: (0, 0)),
        ],
        out_specs=pl.BlockSpec((_BN, _K), lambda i: (i, 0)),
        out_shape=jax.ShapeDtypeStruct((_N, _K), jnp.float32),
        scratch_shapes=[
            pltpu.VMEM((_K, _D), jnp.float32),
            pltpu.VMEM((1, _K), jnp.float32),
            pltpu.VMEM((_BN, _K), jnp.float32),
        ],
    )(X, V)


# speedup vs baseline: 2.1955x; 2.1955x over previous
import jax
import jax.numpy as jnp
from jax.experimental import pallas as pl
from jax.experimental.pallas import tpu as pltpu

_N = 32768
_D = 256
_K = 1024
_BN = 4096
_C = 128


def _kmeans_block(x_ref, v_ref, o_ref, v2s_ref, v2n_ref):
    @pl.when(pl.program_id(0) == 0)
    def _prep():
        v = v_ref[...]
        v2s_ref[...] = v * 2.0
        v2n_ref[...] = jnp.sum(v * v, axis=1)[None, :]

    x = x_ref[...]
    x2 = jnp.sum(x * x, axis=1, keepdims=True)
    v2 = v2n_ref[...]
    xv2 = jax.lax.dot_general(
        x, v2s_ref[...], (((1,), (1,)), ((), ())),
        preferred_element_type=jnp.float32)

    m = (x2 + v2[:, :_C]) - xv2[:, :_C]
    ci = jnp.zeros(m.shape, jnp.int32)
    for i in range(1, _K // _C):
        t = (x2 + v2[:, i * _C:(i + 1) * _C]) - xv2[:, i * _C:(i + 1) * _C]
        lt = t < m
        ci = jnp.where(lt, i, ci)
        m = jnp.minimum(m, t)

    lanes = jax.lax.broadcasted_iota(jnp.int32, m.shape, 1)
    mv = jnp.min(m, axis=1, keepdims=True)
    col = ci * _C + lanes
    first = jnp.min(jnp.where(m == mv, col, _K), axis=1, keepdims=True)
    dmin = jnp.sqrt(jnp.maximum(mv, 1e-12))

    cols = jax.lax.broadcasted_iota(jnp.int32, (_BN, _K), 1)
    o_ref[...] = jnp.where(cols == first, dmin, 0.0)


@jax.jit
def kernel(X, V):
    grid = (_N // _BN,)
    return pl.pallas_call(
        _kmeans_block,
        grid=grid,
        in_specs=[
            pl.BlockSpec((_BN, _D), lambda i: (i, 0)),
            pl.BlockSpec((_K, _D), lambda i: (0, 0)),
        ],
        out_specs=pl.BlockSpec((_BN, _K), lambda i: (i, 0)),
        out_shape=jax.ShapeDtypeStruct((_N, _K), jnp.float32),
        scratch_shapes=[
            pltpu.VMEM((_K, _D), jnp.float32),
            pltpu.VMEM((1, _K), jnp.float32),
        ],
    )(X, V)
